# baseline (speedup 1.0000x reference)
import math

import jax
import jax.numpy as jnp
import numpy as np
from jax.experimental import pallas as pl

STD2 = 0.1
VAR2 = STD2 * STD2
LOG_VAR2 = math.log(VAR2)
IN_DIM = 2048
CTX_DIM = 1024
HID = 256
MAX_AGENT = 6
N_L = 64
N_Q = 16384

QBLK = 1024

_LOGFACT = [float(math.lgamma(k + 1)) for k in range(MAX_AGENT + 1)]
_LOGFACT_COEF = [float(c) for c in np.polyfit(
    np.arange(MAX_AGENT + 1, dtype=np.float64),
    np.array(_LOGFACT, dtype=np.float64), MAX_AGENT)]


def _logfact_poly(v):
    acc = jnp.full_like(v, _LOGFACT_COEF[0])
    for coef in _LOGFACT_COEF[1:]:
        acc = acc * v + coef
    return acc


def _rotl32(x, d):
    return ((x << np.uint32(d)) | (x >> np.uint32(32 - d))).astype(np.uint32)


def _threefry2x32(k0, k1, x0, x1):
    rot = [[13, 15, 26, 6], [17, 29, 16, 24]]
    ks = [np.uint32(k0), np.uint32(k1),
          np.uint32(np.uint32(k0) ^ np.uint32(k1) ^ np.uint32(0x1BD11BDA))]
    x0 = (x0 + ks[0]).astype(np.uint32)
    x1 = (x1 + ks[1]).astype(np.uint32)
    for i in range(5):
        for d in rot[i % 2]:
            x0 = (x0 + x1).astype(np.uint32)
            x1 = _rotl32(x1, d)
            x1 = (x1 ^ x0).astype(np.uint32)
        x0 = (x0 + ks[(i + 1) % 3]).astype(np.uint32)
        x1 = (x1 + ks[(i + 2) % 3] + np.uint32(i + 1)).astype(np.uint32)
    return x0, x1


def _random_bits(k0, k1, n):
    i = np.arange(n, dtype=np.uint64)
    o0, o1 = _threefry2x32(k0, k1, (i >> np.uint64(32)).astype(np.uint32),
                           (i & np.uint64(0xFFFFFFFF)).astype(np.uint32))
    return o0 ^ o1


def _bits_to_unit_float(bits):
    return (((bits >> np.uint32(9)) | np.uint32(0x3F800000)).view(np.float32)
            - np.float32(1.0))


def _erfinv_f32(x):
    x = x.astype(np.float32)
    w = (-np.log((np.float32(1.0) - x) * (np.float32(1.0) + x))
         ).astype(np.float32)
    w1 = (w - np.float32(2.5)).astype(np.float32)
    p = np.full_like(x, np.float32(2.81022636e-08))
    for c in [3.43273939e-07, -3.5233877e-06, -4.39150654e-06, 0.00021858087,
              -0.00125372503, -0.00417768164, 0.246640727, 1.50140941]:
        p = (p * w1 + np.float32(c)).astype(np.float32)
    w2 = (np.sqrt(w).astype(np.float32) - np.float32(3.0)).astype(np.float32)
    q = np.full_like(x, np.float32(-0.000200214257))
    for c in [0.000100950558, 0.00134934322, -0.00367342844, 0.00573950773,
              -0.0076224613, 0.00943887047, 1.00167406, 2.83297682]:
        q = (q * w2 + np.float32(c)).astype(np.float32)
    return np.where(w < np.float32(5.0), (p * x).astype(np.float32),
                    (q * x).astype(np.float32)).astype(np.float32)


def _host_normal(seed, n):
    f = _bits_to_unit_float(_random_bits(0, seed, n))
    lo = np.float32(np.nextafter(np.float32(-1), np.float32(0)))
    hi = np.float32(1.0)
    u = np.maximum(lo, (f * (hi - lo) + lo).astype(np.float32))
    return (np.float32(np.sqrt(np.float32(2.0), dtype=np.float32))
            * _erfinv_f32(u)).astype(np.float32)


_EPS = _host_normal(1234, N_L * HID).reshape(N_L, HID)
_THRESH = np.stack(
    [_bits_to_unit_float(
        _random_bits(*(int(v[0]) for v in _threefry2x32(
            0, 777, np.array([0], np.uint32), np.array([i], np.uint32))),
            N_Q))
     for i in range(1, MAX_AGENT + 1)], axis=1)


def _logfact_lookup(v):
    out = jnp.zeros_like(v)
    for k in range(MAX_AGENT + 1):
        out = out + jnp.where(v == float(k), _LOGFACT[k], 0.0)
    return out


def _vae_kernel(llms_ref, fc1w_ref, fc1b_ref, fc21w_ref, fc21b_ref,
                fc22w_ref, fc22b_ref, fc3w_ref, fc3b_ref, fc4w_ref,
                fc4b_ref, eps_ref, zt_ref, loss_ref):
    llms = llms_ref[...]
    h = jax.nn.relu(
        jnp.dot(llms, fc1w_ref[...], preferred_element_type=jnp.float32)
        + fc1b_ref[...])
    mu = jnp.dot(h, fc21w_ref[...], preferred_element_type=jnp.float32) \
        + fc21b_ref[...]
    log_var = jnp.dot(h, fc22w_ref[...], preferred_element_type=jnp.float32) \
        + fc22b_ref[...]
    std = jnp.exp(0.5 * log_var) * STD2
    z = mu + eps_ref[...] * std
    h2 = jax.nn.relu(
        jnp.dot(z, fc3w_ref[...], preferred_element_type=jnp.float32)
        + fc3b_ref[...])
    x_hat = jnp.dot(h2, fc4w_ref[...], preferred_element_type=jnp.float32) \
        + fc4b_ref[...]
    mse = jnp.mean((x_hat - llms) ** 2)
    kld = -0.5 * jnp.mean(1.0 - LOG_VAR2 + log_var
                          - (mu ** 2 + jnp.exp(log_var)) / VAR2)
    loss_ref[...] = (mse + kld).reshape(1, 1)
    norm = jnp.sqrt(jnp.sum(z * z, axis=1, keepdims=True))
    zn = z / jnp.maximum(norm, 1e-12)
    zt_ref[...] = zn.T


def _route_kernel(ctx_ref, ctxw_ref, ctxb_ref, zt_ref, thr_ref, agent_ref,
                  sel_ref, lp_ref):
    nq = ctx_ref.shape[0]
    ce = jnp.dot(ctx_ref[...], ctxw_ref[...],
                 preferred_element_type=jnp.float32) + ctxb_ref[...]
    n2 = jnp.sum(ce * ce, axis=1, keepdims=True)
    ce = ce * jax.lax.rsqrt(jnp.maximum(n2, 1e-24))
    s = jnp.dot(ce, zt_ref[...], preferred_element_type=jnp.float32)
    e = jnp.exp(s)
    p = e * (1.0 / jnp.sum(e, axis=1, keepdims=True))
    row = jax.lax.broadcasted_iota(jnp.int32, (N_L, N_L), 0)
    col = jax.lax.broadcasted_iota(jnp.int32, (N_L, N_L), 1)
    tri = (row <= col).astype(jnp.float32)
    c = jax.lax.dot(p, tri, precision=jax.lax.Precision.HIGHEST)
    logp = jnp.log(p)
    agent_f = agent_ref[...].astype(jnp.float32)
    amv = (agent_f >= (jax.lax.broadcasted_iota(jnp.int32, (1, 8), 1)
                       .astype(jnp.float32) + 1.0)).astype(jnp.float32)
    masks = []
    acc = jnp.zeros((nq, N_L), jnp.float32)
    for i in range(MAX_AGENT):
        r = thr_ref[:, i:i + 1]
        mask = (c > r).astype(jnp.float32)
        acc = acc + mask * amv[:, i:i + 1]
        masks.append(mask)
    big = jnp.concatenate(masks, axis=1)
    brow = jax.lax.broadcasted_iota(jnp.int32, (MAX_AGENT * N_L, 8), 0)
    bcol = jax.lax.broadcasted_iota(jnp.int32, (MAX_AGENT * N_L, 8), 1)
    bdiag = (brow // N_L == bcol).astype(jnp.float32)
    cnt = jnp.dot(big, bdiag, preferred_element_type=jnp.float32)
    sel_f = jnp.where(cnt < 0.5, 0.0, float(N_L) - cnt)
    sel_ref[...] = sel_f[:, :MAX_AGENT].astype(jnp.int32)
    shifted = jnp.concatenate(
        [jnp.zeros((nq, 1), jnp.float32), acc[:, :N_L - 1]], axis=1)
    sel_llm = acc - shifted
    w = sel_llm * logp - _logfact_poly(sel_llm)
    lp_ref[...] = (_logfact_poly(agent_f)
                   + jnp.sum(w, axis=1, keepdims=True))


def kernel(llms, contexts, agent_num_int, agent_num_float, fc1_w, fc1_b,
           fc21_w, fc21_b, fc22_w, fc22_b, fc3_w, fc3_b, fc4_w, fc4_b,
           ctx_w, ctx_b):
    eps = jnp.asarray(_EPS)
    thresh = jnp.asarray(_THRESH)

    zt, loss = pl.pallas_call(
        _vae_kernel,
        out_shape=(
            jax.ShapeDtypeStruct((HID, N_L), jnp.float32),
            jax.ShapeDtypeStruct((1, 1), jnp.float32),
        ),
    )(llms, fc1_w, fc1_b.reshape(1, HID), fc21_w, fc21_b.reshape(1, HID),
      fc22_w, fc22_b.reshape(1, HID), fc3_w, fc3_b.reshape(1, HID), fc4_w,
      fc4_b.reshape(1, IN_DIM), eps)

    grid = (N_Q // QBLK,)
    sel, lp = pl.pallas_call(
        _route_kernel,
        grid=grid,
        in_specs=[
            pl.BlockSpec((QBLK, CTX_DIM), lambda q: (q, 0)),
            pl.BlockSpec((CTX_DIM, HID), lambda q: (0, 0)),
            pl.BlockSpec((1, HID), lambda q: (0, 0)),
            pl.BlockSpec((HID, N_L), lambda q: (0, 0)),
            pl.BlockSpec((QBLK, MAX_AGENT), lambda q: (q, 0)),
            pl.BlockSpec((QBLK, 1), lambda q: (q, 0)),
        ],
        out_specs=(
            pl.BlockSpec((QBLK, MAX_AGENT), lambda q: (q, 0)),
            pl.BlockSpec((QBLK, 1), lambda q: (q, 0)),
        ),
        out_shape=(
            jax.ShapeDtypeStruct((N_Q, MAX_AGENT), jnp.int32),
            jax.ShapeDtypeStruct((N_Q, 1), jnp.float32),
        ),
    )(contexts, ctx_w, ctx_b.reshape(1, HID), zt, thresh, agent_num_int)

    selected_llm_index = sel.T
    log_probs = lp
    vae_loss = loss.reshape(())
    return (selected_llm_index, log_probs, vae_loss)


# speedup vs baseline: 1.0793x; 1.0793x over previous
import math

import jax
import jax.numpy as jnp
import numpy as np
from jax.experimental import pallas as pl
from jax.experimental.pallas import tpu as pltpu

STD2 = 0.1
VAR2 = STD2 * STD2
LOG_VAR2 = math.log(VAR2)
IN_DIM = 2048
CTX_DIM = 1024
HID = 256
MAX_AGENT = 6
N_L = 64
N_Q = 16384

QBLK = 1024

_LOGFACT = [float(math.lgamma(k + 1)) for k in range(MAX_AGENT + 1)]
_LOGFACT_COEF = [float(c) for c in np.polyfit(
    np.arange(MAX_AGENT + 1, dtype=np.float64),
    np.array(_LOGFACT, dtype=np.float64), MAX_AGENT)]


def _logfact_poly(v):
    acc = jnp.full_like(v, _LOGFACT_COEF[0])
    for coef in _LOGFACT_COEF[1:]:
        acc = acc * v + coef
    return acc


def _rotl32(x, d):
    return ((x << np.uint32(d)) | (x >> np.uint32(32 - d))).astype(np.uint32)


def _threefry2x32(k0, k1, x0, x1):
    rot = [[13, 15, 26, 6], [17, 29, 16, 24]]
    ks = [np.uint32(k0), np.uint32(k1),
          np.uint32(np.uint32(k0) ^ np.uint32(k1) ^ np.uint32(0x1BD11BDA))]
    x0 = (x0 + ks[0]).astype(np.uint32)
    x1 = (x1 + ks[1]).astype(np.uint32)
    for i in range(5):
        for d in rot[i % 2]:
            x0 = (x0 + x1).astype(np.uint32)
            x1 = _rotl32(x1, d)
            x1 = (x1 ^ x0).astype(np.uint32)
        x0 = (x0 + ks[(i + 1) % 3]).astype(np.uint32)
        x1 = (x1 + ks[(i + 2) % 3] + np.uint32(i + 1)).astype(np.uint32)
    return x0, x1


def _random_bits(k0, k1, n):
    i = np.arange(n, dtype=np.uint64)
    o0, o1 = _threefry2x32(k0, k1, (i >> np.uint64(32)).astype(np.uint32),
                           (i & np.uint64(0xFFFFFFFF)).astype(np.uint32))
    return o0 ^ o1


def _bits_to_unit_float(bits):
    return (((bits >> np.uint32(9)) | np.uint32(0x3F800000)).view(np.float32)
            - np.float32(1.0))


def _erfinv_f32(x):
    x = x.astype(np.float32)
    w = (-np.log((np.float32(1.0) - x) * (np.float32(1.0) + x))
         ).astype(np.float32)
    w1 = (w - np.float32(2.5)).astype(np.float32)
    p = np.full_like(x, np.float32(2.81022636e-08))
    for c in [3.43273939e-07, -3.5233877e-06, -4.39150654e-06, 0.00021858087,
              -0.00125372503, -0.00417768164, 0.246640727, 1.50140941]:
        p = (p * w1 + np.float32(c)).astype(np.float32)
    w2 = (np.sqrt(w).astype(np.float32) - np.float32(3.0)).astype(np.float32)
    q = np.full_like(x, np.float32(-0.000200214257))
    for c in [0.000100950558, 0.00134934322, -0.00367342844, 0.00573950773,
              -0.0076224613, 0.00943887047, 1.00167406, 2.83297682]:
        q = (q * w2 + np.float32(c)).astype(np.float32)
    return np.where(w < np.float32(5.0), (p * x).astype(np.float32),
                    (q * x).astype(np.float32)).astype(np.float32)


def _host_normal(seed, n):
    f = _bits_to_unit_float(_random_bits(0, seed, n))
    lo = np.float32(np.nextafter(np.float32(-1), np.float32(0)))
    hi = np.float32(1.0)
    u = np.maximum(lo, (f * (hi - lo) + lo).astype(np.float32))
    return (np.float32(np.sqrt(np.float32(2.0), dtype=np.float32))
            * _erfinv_f32(u)).astype(np.float32)


_EPS = _host_normal(1234, N_L * HID).reshape(N_L, HID)
_THRESH = np.stack(
    [_bits_to_unit_float(
        _random_bits(*(int(v[0]) for v in _threefry2x32(
            0, 777, np.array([0], np.uint32), np.array([i], np.uint32))),
            N_Q))
     for i in range(1, MAX_AGENT + 1)], axis=1)


def _logfact_lookup(v):
    out = jnp.zeros_like(v)
    for k in range(MAX_AGENT + 1):
        out = out + jnp.where(v == float(k), _LOGFACT[k], 0.0)
    return out


def _vae_kernel(llms_ref, fc1w_ref, fc1b_ref, fc21w_ref, fc21b_ref,
                fc22w_ref, fc22b_ref, fc3w_ref, fc3b_ref, fc4w_ref,
                fc4b_ref, eps_ref, zt_ref, loss_ref):
    llms = llms_ref[...]
    h = jax.nn.relu(
        jnp.dot(llms, fc1w_ref[...], preferred_element_type=jnp.float32)
        + fc1b_ref[...])
    mu = jnp.dot(h, fc21w_ref[...], preferred_element_type=jnp.float32) \
        + fc21b_ref[...]
    log_var = jnp.dot(h, fc22w_ref[...], preferred_element_type=jnp.float32) \
        + fc22b_ref[...]
    std = jnp.exp(0.5 * log_var) * STD2
    z = mu + eps_ref[...] * std
    h2 = jax.nn.relu(
        jnp.dot(z, fc3w_ref[...], preferred_element_type=jnp.float32)
        + fc3b_ref[...])
    x_hat = jnp.dot(h2, fc4w_ref[...], preferred_element_type=jnp.float32) \
        + fc4b_ref[...]
    mse = jnp.mean((x_hat - llms) ** 2)
    kld = -0.5 * jnp.mean(1.0 - LOG_VAR2 + log_var
                          - (mu ** 2 + jnp.exp(log_var)) / VAR2)
    loss_ref[...] = (mse + kld).reshape(1, 1)
    norm = jnp.sqrt(jnp.sum(z * z, axis=1, keepdims=True))
    zn = z / jnp.maximum(norm, 1e-12)
    zt_ref[...] = zn.T


def _route_kernel(ctx_ref, ctxw_ref, ctxb_ref, thr_ref, agent_ref,
                  llms_ref, fc1w_ref, fc1b_ref, fc21w_ref, fc21b_ref,
                  fc22w_ref, fc22b_ref, fc3w_ref, fc3b_ref, fc4w_ref,
                  fc4b_ref, eps_ref, sel_ref, lp_ref, loss_ref, zt_ref):
    @pl.when(pl.program_id(0) == 0)
    def _():
        _vae_kernel(llms_ref, fc1w_ref, fc1b_ref, fc21w_ref, fc21b_ref,
                    fc22w_ref, fc22b_ref, fc3w_ref, fc3b_ref, fc4w_ref,
                    fc4b_ref, eps_ref, zt_ref, loss_ref)

    nq = ctx_ref.shape[0]
    ce = jnp.dot(ctx_ref[...], ctxw_ref[...],
                 preferred_element_type=jnp.float32) + ctxb_ref[...]
    norm = jnp.sqrt(jnp.sum(ce * ce, axis=1, keepdims=True))
    ce = ce / jnp.maximum(norm, 1e-12)
    s = jnp.dot(ce, zt_ref[...], preferred_element_type=jnp.float32)
    m = jnp.max(s, axis=1, keepdims=True)
    e = jnp.exp(s - m)
    p = e / jnp.sum(e, axis=1, keepdims=True)
    row = jax.lax.broadcasted_iota(jnp.int32, (N_L, N_L), 0)
    col = jax.lax.broadcasted_iota(jnp.int32, (N_L, N_L), 1)
    tri = (row <= col).astype(jnp.float32)
    c = jax.lax.dot(p, tri, precision=jax.lax.Precision.HIGHEST)
    logp = jnp.log(p)
    agent_f = agent_ref[...].astype(jnp.float32)
    masks = []
    acc = jnp.zeros((nq, N_L), jnp.float32)
    for i in range(MAX_AGENT):
        r = thr_ref[:, i:i + 1]
        mask = (c > r).astype(jnp.float32)
        am = (agent_f >= float(i + 1)).astype(jnp.float32)
        acc = acc + mask * am
        masks.append(mask)
    big = jnp.concatenate(masks, axis=1)
    brow = jax.lax.broadcasted_iota(jnp.int32, (MAX_AGENT * N_L, 8), 0)
    bcol = jax.lax.broadcasted_iota(jnp.int32, (MAX_AGENT * N_L, 8), 1)
    bdiag = (brow // N_L == bcol).astype(jnp.float32)
    cnt = jnp.dot(big, bdiag, preferred_element_type=jnp.float32)
    sel_f = jnp.where(cnt < 0.5, 0.0, float(N_L) - cnt)
    sel_ref[...] = sel_f[:, :MAX_AGENT].astype(jnp.int32)
    shifted = jnp.concatenate(
        [jnp.zeros((nq, 1), jnp.float32), acc[:, :N_L - 1]], axis=1)
    sel_llm = acc - shifted
    w = sel_llm * logp - _logfact_poly(sel_llm)
    lp_ref[...] = (_logfact_poly(agent_f)
                   + jnp.sum(w, axis=1, keepdims=True))


def kernel(llms, contexts, agent_num_int, agent_num_float, fc1_w, fc1_b,
           fc21_w, fc21_b, fc22_w, fc22_b, fc3_w, fc3_b, fc4_w, fc4_b,
           ctx_w, ctx_b):
    eps = jnp.asarray(_EPS)
    thresh = jnp.asarray(_THRESH)

    def _const(shape):
        return pl.BlockSpec(shape, lambda q: tuple(0 for _ in shape))

    grid = (N_Q // QBLK,)
    sel, lp, loss = pl.pallas_call(
        _route_kernel,
        grid=grid,
        in_specs=[
            pl.BlockSpec((QBLK, CTX_DIM), lambda q: (q, 0)),
            _const((CTX_DIM, HID)),
            _const((1, HID)),
            pl.BlockSpec((QBLK, MAX_AGENT), lambda q: (q, 0)),
            pl.BlockSpec((QBLK, 1), lambda q: (q, 0)),
            _const((N_L, IN_DIM)),
            _const((IN_DIM, HID)),
            _const((1, HID)),
            _const((HID, HID)),
            _const((1, HID)),
            _const((HID, HID)),
            _const((1, HID)),
            _const((HID, HID)),
            _const((1, HID)),
            _const((HID, IN_DIM)),
            _const((1, IN_DIM)),
            _const((N_L, HID)),
        ],
        out_specs=(
            pl.BlockSpec((QBLK, MAX_AGENT), lambda q: (q, 0)),
            pl.BlockSpec((QBLK, 1), lambda q: (q, 0)),
            _const((1, 1)),
        ),
        out_shape=(
            jax.ShapeDtypeStruct((N_Q, MAX_AGENT), jnp.int32),
            jax.ShapeDtypeStruct((N_Q, 1), jnp.float32),
            jax.ShapeDtypeStruct((1, 1), jnp.float32),
        ),
        scratch_shapes=[pltpu.VMEM((HID, N_L), jnp.float32)],
    )(contexts, ctx_w, ctx_b.reshape(1, HID), thresh, agent_num_int,
      llms, fc1_w, fc1_b.reshape(1, HID), fc21_w, fc21_b.reshape(1, HID),
      fc22_w, fc22_b.reshape(1, HID), fc3_w, fc3_b.reshape(1, HID),
      fc4_w, fc4_b.reshape(1, IN_DIM), eps)

    selected_llm_index = sel.T
    log_probs = lp
    vae_loss = loss.reshape(())
    return (selected_llm_index, log_probs, vae_loss)


# speedup vs baseline: 1.4421x; 1.3361x over previous
import math

import jax
import jax.numpy as jnp
import numpy as np
from jax.experimental import pallas as pl
from jax.experimental.pallas import tpu as pltpu

STD2 = 0.1
VAR2 = STD2 * STD2
LOG_VAR2 = math.log(VAR2)
IN_DIM = 2048
CTX_DIM = 1024
HID = 256
MAX_AGENT = 6
N_L = 64
N_Q = 16384

QBLK = 2048

_LOGFACT = [float(math.lgamma(k + 1)) for k in range(MAX_AGENT + 1)]
_LOGFACT_COEF = [float(c) for c in np.polyfit(
    np.arange(MAX_AGENT + 1, dtype=np.float64),
    np.array(_LOGFACT, dtype=np.float64), MAX_AGENT)]


def _logfact_poly(v):
    acc = jnp.full_like(v, _LOGFACT_COEF[0])
    for coef in _LOGFACT_COEF[1:]:
        acc = acc * v + coef
    return acc


def _rotl32(x, d):
    return ((x << np.uint32(d)) | (x >> np.uint32(32 - d))).astype(np.uint32)


def _threefry2x32(k0, k1, x0, x1):
    rot = [[13, 15, 26, 6], [17, 29, 16, 24]]
    ks = [np.uint32(k0), np.uint32(k1),
          np.uint32(np.uint32(k0) ^ np.uint32(k1) ^ np.uint32(0x1BD11BDA))]
    x0 = (x0 + ks[0]).astype(np.uint32)
    x1 = (x1 + ks[1]).astype(np.uint32)
    for i in range(5):
        for d in rot[i % 2]:
            x0 = (x0 + x1).astype(np.uint32)
            x1 = _rotl32(x1, d)
            x1 = (x1 ^ x0).astype(np.uint32)
        x0 = (x0 + ks[(i + 1) % 3]).astype(np.uint32)
        x1 = (x1 + ks[(i + 2) % 3] + np.uint32(i + 1)).astype(np.uint32)
    return x0, x1


def _random_bits(k0, k1, n):
    i = np.arange(n, dtype=np.uint64)
    o0, o1 = _threefry2x32(k0, k1, (i >> np.uint64(32)).astype(np.uint32),
                           (i & np.uint64(0xFFFFFFFF)).astype(np.uint32))
    return o0 ^ o1


def _bits_to_unit_float(bits):
    return (((bits >> np.uint32(9)) | np.uint32(0x3F800000)).view(np.float32)
            - np.float32(1.0))


def _erfinv_f32(x):
    x = x.astype(np.float32)
    w = (-np.log((np.float32(1.0) - x) * (np.float32(1.0) + x))
         ).astype(np.float32)
    w1 = (w - np.float32(2.5)).astype(np.float32)
    p = np.full_like(x, np.float32(2.81022636e-08))
    for c in [3.43273939e-07, -3.5233877e-06, -4.39150654e-06, 0.00021858087,
              -0.00125372503, -0.00417768164, 0.246640727, 1.50140941]:
        p = (p * w1 + np.float32(c)).astype(np.float32)
    w2 = (np.sqrt(w).astype(np.float32) - np.float32(3.0)).astype(np.float32)
    q = np.full_like(x, np.float32(-0.000200214257))
    for c in [0.000100950558, 0.00134934322, -0.00367342844, 0.00573950773,
              -0.0076224613, 0.00943887047, 1.00167406, 2.83297682]:
        q = (q * w2 + np.float32(c)).astype(np.float32)
    return np.where(w < np.float32(5.0), (p * x).astype(np.float32),
                    (q * x).astype(np.float32)).astype(np.float32)


def _host_normal(seed, n):
    f = _bits_to_unit_float(_random_bits(0, seed, n))
    lo = np.float32(np.nextafter(np.float32(-1), np.float32(0)))
    hi = np.float32(1.0)
    u = np.maximum(lo, (f * (hi - lo) + lo).astype(np.float32))
    return (np.float32(np.sqrt(np.float32(2.0), dtype=np.float32))
            * _erfinv_f32(u)).astype(np.float32)


_EPS = _host_normal(1234, N_L * HID).reshape(N_L, HID)
_THRESH = np.stack(
    [_bits_to_unit_float(
        _random_bits(*(int(v[0]) for v in _threefry2x32(
            0, 777, np.array([0], np.uint32), np.array([i], np.uint32))),
            N_Q))
     for i in range(1, MAX_AGENT + 1)], axis=1)


def _logfact_lookup(v):
    out = jnp.zeros_like(v)
    for k in range(MAX_AGENT + 1):
        out = out + jnp.where(v == float(k), _LOGFACT[k], 0.0)
    return out


def _vae_kernel(llms_ref, fc1w_ref, fc1b_ref, fc21w_ref, fc21b_ref,
                fc22w_ref, fc22b_ref, fc3w_ref, fc3b_ref, fc4w_ref,
                fc4b_ref, eps_ref, zt_ref, loss_ref):
    llms = llms_ref[...]
    h = jax.nn.relu(
        jnp.dot(llms, fc1w_ref[...], preferred_element_type=jnp.float32)
        + fc1b_ref[...])
    mu = jnp.dot(h, fc21w_ref[...], preferred_element_type=jnp.float32) \
        + fc21b_ref[...]
    log_var = jnp.dot(h, fc22w_ref[...], preferred_element_type=jnp.float32) \
        + fc22b_ref[...]
    std = jnp.exp(0.5 * log_var) * STD2
    z = mu + eps_ref[...] * std
    h2 = jax.nn.relu(
        jnp.dot(z, fc3w_ref[...], preferred_element_type=jnp.float32)
        + fc3b_ref[...])
    x_hat = jnp.dot(h2, fc4w_ref[...], preferred_element_type=jnp.float32) \
        + fc4b_ref[...]
    mse = jnp.mean((x_hat - llms) ** 2)
    kld = -0.5 * jnp.mean(1.0 - LOG_VAR2 + log_var
                          - (mu ** 2 + jnp.exp(log_var)) / VAR2)
    loss_ref[...] = (mse + kld).reshape(1, 1)
    norm = jnp.sqrt(jnp.sum(z * z, axis=1, keepdims=True))
    zn = z / jnp.maximum(norm, 1e-12)
    zt_ref[...] = zn.T


def _route_kernel(ctx_ref, ctxw_ref, ctxb_ref, thr_ref, agent_ref,
                  llms_ref, fc1w_ref, fc1b_ref, fc21w_ref, fc21b_ref,
                  fc22w_ref, fc22b_ref, fc3w_ref, fc3b_ref, fc4w_ref,
                  fc4b_ref, eps_ref, sel_ref, lp_ref, loss_ref, zt_ref):
    @pl.when(pl.program_id(0) == 0)
    def _():
        _vae_kernel(llms_ref, fc1w_ref, fc1b_ref, fc21w_ref, fc21b_ref,
                    fc22w_ref, fc22b_ref, fc3w_ref, fc3b_ref, fc4w_ref,
                    fc4b_ref, eps_ref, zt_ref, loss_ref)

    nq = ctx_ref.shape[0]
    ce = jnp.dot(ctx_ref[...], ctxw_ref[...],
                 preferred_element_type=jnp.float32) + ctxb_ref[...]
    norm = jnp.sqrt(jnp.sum(ce * ce, axis=1, keepdims=True))
    ce = ce / jnp.maximum(norm, 1e-12)
    s = jnp.dot(ce, zt_ref[...], preferred_element_type=jnp.float32)
    m = jnp.max(s, axis=1, keepdims=True)
    e = jnp.exp(s - m)
    p = e / jnp.sum(e, axis=1, keepdims=True)
    row = jax.lax.broadcasted_iota(jnp.int32, (N_L, N_L), 0)
    col = jax.lax.broadcasted_iota(jnp.int32, (N_L, N_L), 1)
    tri = (row <= col).astype(jnp.float32)
    c = jax.lax.dot(p, tri, precision=jax.lax.Precision.HIGHEST)
    logp = jnp.log(p)
    agent_f = agent_ref[...].astype(jnp.float32)
    masks = []
    acc = jnp.zeros((nq, N_L), jnp.float32)
    for i in range(MAX_AGENT):
        r = thr_ref[:, i:i + 1]
        mask = (c > r).astype(jnp.float32)
        am = (agent_f >= float(i + 1)).astype(jnp.float32)
        acc = acc + mask * am
        masks.append(mask)
    big = jnp.concatenate(masks, axis=1)
    brow = jax.lax.broadcasted_iota(jnp.int32, (MAX_AGENT * N_L, 8), 0)
    bcol = jax.lax.broadcasted_iota(jnp.int32, (MAX_AGENT * N_L, 8), 1)
    bdiag = (brow // N_L == bcol).astype(jnp.float32)
    cnt = jnp.dot(big, bdiag, preferred_element_type=jnp.float32)
    sel_f = jnp.where(cnt < 0.5, 0.0, float(N_L) - cnt)
    sel_ref[...] = sel_f[:, :MAX_AGENT].astype(jnp.int32)
    shifted = jnp.concatenate(
        [jnp.zeros((nq, 1), jnp.float32), acc[:, :N_L - 1]], axis=1)
    sel_llm = acc - shifted
    w = sel_llm * logp - _logfact_poly(sel_llm)
    lp_ref[...] = (_logfact_poly(agent_f)
                   + jnp.sum(w, axis=1, keepdims=True))


def kernel(llms, contexts, agent_num_int, agent_num_float, fc1_w, fc1_b,
           fc21_w, fc21_b, fc22_w, fc22_b, fc3_w, fc3_b, fc4_w, fc4_b,
           ctx_w, ctx_b):
    eps = jnp.asarray(_EPS)
    thresh = jnp.asarray(_THRESH)

    def _const(shape):
        return pl.BlockSpec(shape, lambda q: tuple(0 for _ in shape))

    grid = (N_Q // QBLK,)
    sel, lp, loss = pl.pallas_call(
        _route_kernel,
        grid=grid,
        in_specs=[
            pl.BlockSpec((QBLK, CTX_DIM), lambda q: (q, 0)),
            _const((CTX_DIM, HID)),
            _const((1, HID)),
            pl.BlockSpec((QBLK, MAX_AGENT), lambda q: (q, 0)),
            pl.BlockSpec((QBLK, 1), lambda q: (q, 0)),
            _const((N_L, IN_DIM)),
            _const((IN_DIM, HID)),
            _const((1, HID)),
            _const((HID, HID)),
            _const((1, HID)),
            _const((HID, HID)),
            _const((1, HID)),
            _const((HID, HID)),
            _const((1, HID)),
            _const((HID, IN_DIM)),
            _const((1, IN_DIM)),
            _const((N_L, HID)),
        ],
        out_specs=(
            pl.BlockSpec((QBLK, MAX_AGENT), lambda q: (q, 0)),
            pl.BlockSpec((QBLK, 1), lambda q: (q, 0)),
            _const((1, 1)),
        ),
        out_shape=(
            jax.ShapeDtypeStruct((N_Q, MAX_AGENT), jnp.int32),
            jax.ShapeDtypeStruct((N_Q, 1), jnp.float32),
            jax.ShapeDtypeStruct((1, 1), jnp.float32),
        ),
        scratch_shapes=[pltpu.VMEM((HID, N_L), jnp.float32)],
    )(contexts, ctx_w, ctx_b.reshape(1, HID), thresh, agent_num_int,
      llms, fc1_w, fc1_b.reshape(1, HID), fc21_w, fc21_b.reshape(1, HID),
      fc22_w, fc22_b.reshape(1, HID), fc3_w, fc3_b.reshape(1, HID),
      fc4_w, fc4_b.reshape(1, IN_DIM), eps)

    selected_llm_index = sel.T
    log_probs = lp
    vae_loss = loss.reshape(())
    return (selected_llm_index, log_probs, vae_loss)
